# baseline (speedup 1.0000x reference)
import jax
import jax.numpy as jnp
from jax import lax
from jax.experimental import pallas as pl
from jax.experimental.pallas import tpu as pltpu
from jax.experimental.pallas import tpu_sc as plsc

_B = 128
_T = 196
_C = 768
_P = 1024
_K = 8
_L = 8
_KL = _K * _L
_BB = 16
_GRID = _B // _BB
_NC = 2
_NS = 16
_NW = _NC * _NS
_KPW = _KL // _NW


def _main_body(x_ref, pk_ref, simo_ref, idx_ref, i64_ref, sim_ref, rsum_ref,
               pet_ref, pn_ref, pkv_ref, wsem, psem):
    step = pl.program_id(0)
    tail_cp = pltpu.make_async_copy(
        x_ref, pet_ref.at[pl.ds(_KL, _T), pl.ds(step * _BB, _BB), :], wsem)
    tail_cp.start()

    @pl.when(step == 0)
    def _():
        pltpu.make_async_copy(pk_ref, pkv_ref, psem).start()
        pltpu.make_async_copy(pk_ref, pkv_ref, psem).wait()
        pk = pkv_ref[...]
        ssq0 = jnp.sum(pk * pk, axis=1, keepdims=True)
        pn_ref[...] = pk * lax.rsqrt(jnp.maximum(ssq0, 1e-12))

    x = x_ref[...]
    xm = jnp.mean(x, axis=0)
    ssq = jnp.sum(xm * xm, axis=1, keepdims=True)
    xn = xm * lax.rsqrt(jnp.maximum(ssq, 1e-12))
    pn = pn_ref[...]
    s = lax.dot_general(xn, pn, (((1,), (1,)), ((), ())),
                        preferred_element_type=jnp.float32)
    simo_ref[...] = s
    iota = lax.broadcasted_iota(jnp.int32, (_BB, _P), 1)
    work = s
    cols = []
    for _ in range(_K):
        m = jnp.max(work, axis=1, keepdims=True)
        cand = jnp.where(work == m, iota, jnp.int32(2**30))
        a = jnp.min(cand, axis=1)
        cols.append(a[:, None])
        work = jnp.where(iota == a[:, None], -jnp.inf, work)
    idx = jnp.concatenate(cols, axis=1)
    idx_ref[...] = idx
    r64 = []
    for k in range(_K):
        av = cols[k][:, 0] * _L
        for j in range(_L):
            r64.append((av + j)[None, :])
    i64_ref[0] = jnp.concatenate(r64, axis=0)
    iota3 = lax.broadcasted_iota(jnp.int32, (_BB, _K, _P), 2)
    oh = (iota3 == idx[:, :, None]).astype(jnp.float32)
    bkn = jnp.dot(oh.reshape(_BB * _K, _P), pn,
                  preferred_element_type=jnp.float32).reshape(_BB, _K, _C)
    simv = bkn * xn[:, None, :]
    sim_ref[...] = simv
    part = jnp.sum(simv) * (1.0 / _B)

    @pl.when(step == 0)
    def _():
        rsum_ref[0, 0] = part

    @pl.when(step > 0)
    def _():
        rsum_ref[0, 0] += part

    tail_cp.wait()


def _gather_body(ptab_ref, idx64t_ref, pet_ref, bp_ref, idxv, rows0, rows1,
                 sem, wsem):
    wid = lax.axis_index("s") * _NC + lax.axis_index("c")
    hb = _B // 2
    bufs = (rows0, rows1)
    stage = []
    for t in range(_KPW):
        for g in range(_GRID):
            stage.append(pltpu.async_copy(
                idx64t_ref.at[g, wid * _KPW + t],
                idxv.at[t, pl.ds(g * _BB, _BB)], sem))
    for h_ in stage:
        h_.wait()

    def _gather(c, buf):
        sel = idxv.at[c // 2, pl.ds((c % 2) * hb, hb)]
        return pltpu.async_copy(ptab_ref.at[sel], buf, sem)

    nchunk = 2 * _KPW
    pending = _gather(0, bufs[0])
    writes = [None, None]
    for c in range(nchunk):
        pending.wait()
        if c + 1 < nchunk:
            if writes[(c + 1) % 2] is not None:
                for w in writes[(c + 1) % 2]:
                    w.wait()
            pending = _gather(c + 1, bufs[(c + 1) % 2])
        kk = wid * _KPW + c // 2
        h = (c % 2) * hb
        writes[c % 2] = (
            pltpu.async_copy(bufs[c % 2], pet_ref.at[kk, pl.ds(h, hb)], wsem),
            pltpu.async_copy(bufs[c % 2], bp_ref.at[pl.ds(h, hb), kk], wsem),
        )
    for ws in writes:
        if ws is not None:
            for w in ws:
                w.wait()


def kernel(x_embed, prompt, prompt_key):
    f32 = jnp.float32
    x_t = jnp.transpose(x_embed, (1, 0, 2))
    similarity, idx, idx64t3, sim, rsum, pet_partial = pl.pallas_call(
        _main_body,
        grid=(_GRID,),
        in_specs=[
            pl.BlockSpec((_T, _BB, _C), lambda i: (0, i, 0)),
            pl.BlockSpec(memory_space=pl.ANY),
        ],
        out_specs=[
            pl.BlockSpec((_BB, _P), lambda i: (i, 0)),
            pl.BlockSpec((_BB, _K), lambda i: (i, 0)),
            pl.BlockSpec((1, _KL, _BB), lambda i: (i, 0, 0)),
            pl.BlockSpec((_BB, _K, _C), lambda i: (i, 0, 0)),
            pl.BlockSpec((1, 1), lambda i: (0, 0), memory_space=pltpu.SMEM),
            pl.BlockSpec(memory_space=pl.ANY),
        ],
        out_shape=[
            jax.ShapeDtypeStruct((_B, _P), f32),
            jax.ShapeDtypeStruct((_B, _K), jnp.int32),
            jax.ShapeDtypeStruct((_GRID, _KL, _BB), jnp.int32),
            jax.ShapeDtypeStruct((_B, _K, _C), f32),
            jax.ShapeDtypeStruct((1, 1), f32),
            jax.ShapeDtypeStruct((_KL + _T, _B, _C), f32),
        ],
        scratch_shapes=[pltpu.VMEM((_P, _C), f32), pltpu.VMEM((_P, _C), f32),
                        pltpu.SemaphoreType.DMA, pltpu.SemaphoreType.DMA],
    )(x_t, prompt_key)

    ptab = prompt.reshape(_P * _L, _C)

    pet_ref = jax.new_ref(pet_partial)
    mesh = plsc.VectorSubcoreMesh(core_axis_name="c", subcore_axis_name="s",
                                  num_cores=_NC, num_subcores=_NS)
    bp = pl.kernel(
        _gather_body,
        out_type=jax.ShapeDtypeStruct((_B, _KL, _C), f32),
        mesh=mesh,
        scratch_types=[
            pltpu.VMEM((_KPW, _B), jnp.int32),
            pltpu.VMEM((_B // 2, _C), f32),
            pltpu.VMEM((_B // 2, _C), f32),
            pltpu.SemaphoreType.DMA,
            pltpu.SemaphoreType.DMA,
        ],
    )(ptab, idx64t3, pet_ref)
    pe = jnp.transpose(pet_ref[...], (1, 0, 2))

    return (pe, similarity, rsum.reshape(()), idx, bp, sim)


# speedup vs baseline: 1.0484x; 1.0484x over previous
import jax
import jax.numpy as jnp
from jax import lax
from jax.experimental import pallas as pl
from jax.experimental.pallas import tpu as pltpu
from jax.experimental.pallas import tpu_sc as plsc

_B = 128
_T = 196
_C = 768
_P = 1024
_K = 8
_L = 8
_KL = _K * _L
_BB = 16
_GRID = _B // _BB
_NC = 2
_NS = 16
_NW = _NC * _NS
_KPW = _KL // _NW


def _main_body(x_ref, pk_ref, simo_ref, idx_ref, i64_ref, xn_ref,
               pet_ref, pn_ref, wsem):
    step = pl.program_id(0)
    tail_cp = pltpu.make_async_copy(
        x_ref, pet_ref.at[pl.ds(_KL, _T), pl.ds(step * _BB, _BB), :], wsem)
    tail_cp.start()

    @pl.when(step == 0)
    def _():
        pk = pk_ref[...]
        ssq0 = jnp.sum(pk * pk, axis=1, keepdims=True)
        pn_ref[...] = pk * lax.rsqrt(jnp.maximum(ssq0, 1e-12))

    x = x_ref[...]
    xm = jnp.mean(x, axis=0)
    ssq = jnp.sum(xm * xm, axis=1, keepdims=True)
    xn = xm * lax.rsqrt(jnp.maximum(ssq, 1e-12))
    pn = pn_ref[...]
    s = lax.dot_general(xn, pn, (((1,), (1,)), ((), ())),
                        preferred_element_type=jnp.float32)
    simo_ref[...] = s
    iota = lax.broadcasted_iota(jnp.int32, (_BB, _P), 1)
    work = s
    cols = []
    for _ in range(_K):
        m = jnp.max(work, axis=1, keepdims=True)
        cand = jnp.where(work == m, iota, jnp.int32(2**30))
        a = jnp.min(cand, axis=1)
        cols.append(a[:, None])
        work = jnp.where(iota == a[:, None], -jnp.inf, work)
    idx = jnp.concatenate(cols, axis=1)
    idx_ref[...] = idx
    r64 = []
    for k in range(_K):
        av = cols[k][:, 0] * _L
        for j in range(_L):
            r64.append((av + j)[None, :])
    i64_ref[0] = jnp.concatenate(r64, axis=0)
    xn_ref[...] = xn
    tail_cp.wait()


def _sim_body(pk_ref, xn_ref, idx_ref, sim_ref, rsum_ref):
    pk = pk_ref[...]
    ssq0 = jnp.sum(pk * pk, axis=1, keepdims=True)
    pn = pk * lax.rsqrt(jnp.maximum(ssq0, 1e-12))
    xn = xn_ref[...]
    idx = idx_ref[...]
    iota3 = lax.broadcasted_iota(jnp.int32, (_B, _K, _P), 2)
    oh = (iota3 == idx[:, :, None]).astype(jnp.float32)
    bkn = jnp.dot(oh.reshape(_B * _K, _P), pn,
                  preferred_element_type=jnp.float32).reshape(_B, _K, _C)
    simv = bkn * xn[:, None, :]
    sim_ref[...] = simv
    rsum_ref[0, 0] = jnp.sum(simv) * (1.0 / _B)


def _gather_body(ptab_ref, idx64t_ref, pet_ref, bp_ref, idxv, rows0, rows1,
                 sem, wsem):
    wid = lax.axis_index("s") * _NC + lax.axis_index("c")
    hb = _B // 2
    bufs = (rows0, rows1)
    stage = []
    for t in range(_KPW):
        for g in range(_GRID):
            stage.append(pltpu.async_copy(
                idx64t_ref.at[g, wid * _KPW + t],
                idxv.at[t, pl.ds(g * _BB, _BB)], sem))
    for h_ in stage:
        h_.wait()

    def _gather(c, buf):
        sel = idxv.at[c // 2, pl.ds((c % 2) * hb, hb)]
        return pltpu.async_copy(ptab_ref.at[sel], buf, sem)

    nchunk = 2 * _KPW
    pending = _gather(0, bufs[0])
    writes = [None, None]
    for c in range(nchunk):
        pending.wait()
        if c + 1 < nchunk:
            if writes[(c + 1) % 2] is not None:
                for w in writes[(c + 1) % 2]:
                    w.wait()
            pending = _gather(c + 1, bufs[(c + 1) % 2])
        kk = wid * _KPW + c // 2
        h = (c % 2) * hb
        writes[c % 2] = (
            pltpu.async_copy(bufs[c % 2], pet_ref.at[kk, pl.ds(h, hb)], wsem),
            pltpu.async_copy(bufs[c % 2], bp_ref.at[pl.ds(h, hb), kk], wsem),
        )
    for ws in writes:
        if ws is not None:
            for w in ws:
                w.wait()


def kernel(x_embed, prompt, prompt_key):
    f32 = jnp.float32
    x_t = jnp.transpose(x_embed, (1, 0, 2))
    similarity, idx, idx64t3, xn, pet_partial = pl.pallas_call(
        _main_body,
        grid=(_GRID,),
        in_specs=[
            pl.BlockSpec((_T, _BB, _C), lambda i: (0, i, 0)),
            pl.BlockSpec((_P, _C), lambda i: (0, 0)),
        ],
        out_specs=[
            pl.BlockSpec((_BB, _P), lambda i: (i, 0)),
            pl.BlockSpec((_BB, _K), lambda i: (i, 0)),
            pl.BlockSpec((1, _KL, _BB), lambda i: (i, 0, 0)),
            pl.BlockSpec((_BB, _C), lambda i: (i, 0)),
            pl.BlockSpec(memory_space=pl.ANY),
        ],
        out_shape=[
            jax.ShapeDtypeStruct((_B, _P), f32),
            jax.ShapeDtypeStruct((_B, _K), jnp.int32),
            jax.ShapeDtypeStruct((_GRID, _KL, _BB), jnp.int32),
            jax.ShapeDtypeStruct((_B, _C), f32),
            jax.ShapeDtypeStruct((_KL + _T, _B, _C), f32),
        ],
        scratch_shapes=[pltpu.VMEM((_P, _C), f32), pltpu.SemaphoreType.DMA],
    )(x_t, prompt_key)

    sim, rsum = pl.pallas_call(
        _sim_body,
        out_shape=[
            jax.ShapeDtypeStruct((_B, _K, _C), f32),
            jax.ShapeDtypeStruct((1, 1), f32),
        ],
        out_specs=[
            pl.BlockSpec((_B, _K, _C), lambda: (0, 0, 0)),
            pl.BlockSpec((1, 1), lambda: (0, 0), memory_space=pltpu.SMEM),
        ],
    )(prompt_key, xn, idx)

    ptab = prompt.reshape(_P * _L, _C)

    pet_ref = jax.new_ref(pet_partial)
    mesh = plsc.VectorSubcoreMesh(core_axis_name="c", subcore_axis_name="s",
                                  num_cores=_NC, num_subcores=_NS)
    bp = pl.kernel(
        _gather_body,
        out_type=jax.ShapeDtypeStruct((_B, _KL, _C), f32),
        mesh=mesh,
        scratch_types=[
            pltpu.VMEM((_KPW, _B), jnp.int32),
            pltpu.VMEM((_B // 2, _C), f32),
            pltpu.VMEM((_B // 2, _C), f32),
            pltpu.SemaphoreType.DMA,
            pltpu.SemaphoreType.DMA,
        ],
    )(ptab, idx64t3, pet_ref)
    pe = jnp.transpose(pet_ref[...], (1, 0, 2))

    return (pe, similarity, rsum.reshape(()), idx, bp, sim)
